# baseline (device time: 106394 ns/iter reference)
import jax
import jax.numpy as jnp
from jax import lax
from jax.experimental import pallas as pl
from jax.experimental.pallas import tpu as pltpu

N_DEV = 4
M_LOC = 1024
HALF = 512
Q = 256
K = 4096
KC = 512
N_LOC = 2048


def kernel(x, w_mat, scale_x, scale_w):
    def body(x_hbm, w_hbm, sx_ref, sw_ref, out_hbm,
             xq, bufL, bufR, bufO, wq, wtmp, xtmp, stage,
             send_sems, recv_sems, wdma_sems, xdma_sems, out_sems):
        me = lax.axis_index("i")
        left = lax.rem(me + N_DEV - 1, N_DEV)
        right = lax.rem(me + 1, N_DEV)
        opp = lax.rem(me + 2, N_DEV)

        def quarter(ref, q):
            return ref.at[pl.ds(q * Q, Q), :]

        def rdma(src, dst, i, dev):
            return pltpu.make_async_remote_copy(
                src_ref=src, dst_ref=dst,
                send_sem=send_sems.at[i], recv_sem=recv_sems.at[i],
                device_id=(dev,), device_id_type=pl.DeviceIdType.MESH)

        def hop0(q):
            return (rdma(quarter(xq, q), quarter(bufL, q), q, right),
                    rdma(quarter(xq, q), quarter(bufR, q), 4 + q, left))

        fwd_r = [rdma(quarter(bufL, j), quarter(bufO, j), 8 + j, right)
                 for j in range(2)]
        fwd_l = [rdma(quarter(bufR, 2 + j), quarter(bufO, 2 + j), 10 + j, left)
                 for j in range(2)]

        def xcopy(q, slot):
            return pltpu.make_async_copy(
                x_hbm.at[pl.ds(q * Q, Q), :], xtmp.at[slot],
                xdma_sems.at[slot])

        xcopy(0, 0).start()
        xcopy(1, 1).start()


        for q in range(4):
            xcopy(q, q % 2).wait()
            xq[pl.ds(q * Q, Q), :] = xtmp[q % 2].astype(jnp.float8_e4m3fn)
            if q + 2 < 4:
                xcopy(q + 2, q % 2).start()
            ra, la = hop0(q)
            ra.start()
            la.start()

        def wcopy(k, slot):
            return pltpu.make_async_copy(
                w_hbm.at[pl.ds(k * KC, KC), pl.ds(me * N_LOC, N_LOC)],
                wtmp.at[slot], wdma_sems.at[slot])

        wcopy(0, 0).start()
        wcopy(1, 1).start()
        for k in range(K // KC):
            wcopy(k, k % 2).wait()
            wq[pl.ds(k * KC, KC), :] = wtmp[k % 2].astype(jnp.float8_e5m2)
            if k + 2 < K // KC:
                wcopy(k + 2, k % 2).start()

        scale = sx_ref[0] * sw_ref[0]

        def mm(src, rows, row0):
            acc = lax.dot_general(
                src[pl.ds(row0, rows), :], wq[...],
                (((1,), (0,)), ((), ())),
                preferred_element_type=jnp.float32,
            )
            return acc * scale

        pending = [None, None]
        state = [0]

        def produce(origin, row0, rows, src):
            slot = state[0] % 2
            state[0] += 1
            if pending[slot] is not None:
                pending[slot].wait()
            stage[slot, pl.ds(0, rows), :] = mm(src, rows, row0)
            cp = pltpu.make_async_copy(
                stage.at[slot, pl.ds(0, rows), :],
                out_hbm.at[pl.ds(origin * M_LOC + row0, rows), :],
                out_sems.at[slot])
            cp.start()
            pending[slot] = cp

        produce(me, 0, HALF, xq)
        produce(me, HALF, HALF, xq)

        recv_r = [rdma(quarter(xq, q), quarter(bufL, q), q, right)
                  for q in range(4)]
        recv_l = [rdma(quarter(xq, q), quarter(bufR, q), 4 + q, left)
                  for q in range(4)]

        recv_r[0].wait_recv()
        fwd_r[0].start()
        recv_r[1].wait_recv()
        fwd_r[1].start()
        recv_l[0].wait_recv()
        recv_l[1].wait_recv()
        produce(right, 0, HALF, bufR)
        recv_l[2].wait_recv()
        fwd_l[0].start()
        recv_l[3].wait_recv()
        fwd_l[1].start()
        produce(right, HALF, HALF, bufR)
        recv_r[2].wait_recv()
        recv_r[3].wait_recv()
        produce(left, 0, HALF, bufL)
        produce(left, HALF, HALF, bufL)

        fwd_r[0].wait_recv()
        produce(opp, 0, Q, bufO)
        fwd_l[0].wait_recv()
        produce(opp, 2 * Q, Q, bufO)
        fwd_r[1].wait_recv()
        produce(opp, Q, Q, bufO)
        fwd_l[1].wait_recv()
        produce(opp, 3 * Q, Q, bufO)

        for q in range(4):
            recv_r[q].wait_send()
            recv_l[q].wait_send()
        fwd_r[0].wait_send()
        fwd_r[1].wait_send()
        fwd_l[0].wait_send()
        fwd_l[1].wait_send()
        pending[0].wait()
        pending[1].wait()

    fp8full = pltpu.VMEM((M_LOC, K), jnp.float8_e4m3fn)
    return pl.pallas_call(
        body,
        out_shape=jax.ShapeDtypeStruct((N_DEV * M_LOC, N_LOC), jnp.float32),
        in_specs=[
            pl.BlockSpec(memory_space=pl.ANY),
            pl.BlockSpec(memory_space=pl.ANY),
            pl.BlockSpec(memory_space=pltpu.SMEM),
            pl.BlockSpec(memory_space=pltpu.SMEM),
        ],
        out_specs=pl.BlockSpec(memory_space=pl.ANY),
        scratch_shapes=[
            fp8full,
            fp8full, fp8full, fp8full,
            pltpu.VMEM((K, N_LOC), jnp.float8_e5m2),
            pltpu.VMEM((2, KC, N_LOC), jnp.float32),
            pltpu.VMEM((2, Q, K), jnp.float32),
            pltpu.VMEM((2, HALF, N_LOC), jnp.float32),
            pltpu.SemaphoreType.DMA((12,)),
            pltpu.SemaphoreType.DMA((12,)),
            pltpu.SemaphoreType.DMA((2,)),
            pltpu.SemaphoreType.DMA((2,)),
            pltpu.SemaphoreType.DMA((2,)),
        ],
        compiler_params=pltpu.CompilerParams(
            vmem_limit_bytes=100 * 1024 * 1024,
        ),
    )(x, w_mat, scale_x, scale_w)


# device time: 101501 ns/iter; 1.0482x vs baseline; 1.0482x over previous
import jax
import jax.numpy as jnp
from jax import lax
from jax.experimental import pallas as pl
from jax.experimental.pallas import tpu as pltpu

N_DEV = 4
M_LOC = 1024
HALF = 512
Q = 256
E = 128
K = 4096
KC = 512
N_LOC = 2048


def kernel(x, w_mat, scale_x, scale_w):
    def body(x_hbm, w_hbm, sx_ref, sw_ref, out_hbm,
             xq, bufL, bufR, bufO, wq, wtmp, xtmp, stage,
             send_sems, recv_sems, wdma_sems, xdma_sems, out_sems):
        me = lax.axis_index("i")
        left = lax.rem(me + N_DEV - 1, N_DEV)
        right = lax.rem(me + 1, N_DEV)
        opp = lax.rem(me + 2, N_DEV)

        def rows(ref, row0, n):
            return ref.at[pl.ds(row0, n), :]

        def rdma(src, dst, i, dev):
            return pltpu.make_async_remote_copy(
                src_ref=src, dst_ref=dst,
                send_sem=send_sems.at[i], recv_sem=recv_sems.at[i],
                device_id=(dev,), device_id_type=pl.DeviceIdType.MESH)

        def hop0(q):
            return (rdma(rows(xq, q * Q, Q), rows(bufL, q * Q, Q), q, right),
                    rdma(rows(xq, q * Q, Q), rows(bufR, q * Q, Q), 4 + q, left))

        FWD_R = [(0, Q), (Q, E), (Q + E, E)]
        FWD_L = [(HALF, Q), (HALF + Q, E), (HALF + Q + E, E)]
        fwd_r = [rdma(rows(bufL, r0, n), rows(bufO, r0, n), 8 + j, right)
                 for j, (r0, n) in enumerate(FWD_R)]
        fwd_l = [rdma(rows(bufR, r0, n), rows(bufO, r0, n), 11 + j, left)
                 for j, (r0, n) in enumerate(FWD_L)]

        def xcopy(q, slot):
            return pltpu.make_async_copy(
                x_hbm.at[pl.ds(q * Q, Q), :], xtmp.at[slot],
                xdma_sems.at[slot])

        xcopy(0, 0).start()
        xcopy(1, 1).start()
        xcopy(0, 0).wait()
        xq[pl.ds(0, Q), :] = xtmp[0].astype(jnp.float8_e4m3fn)
        xcopy(2, 0).start()

        barrier_sem = pltpu.get_barrier_semaphore()
        for nbr in (left, right):
            pl.semaphore_signal(barrier_sem, inc=1, device_id=(nbr,),
                                device_id_type=pl.DeviceIdType.MESH)
        pl.semaphore_wait(barrier_sem, 2)

        ra, la = hop0(0)
        ra.start()
        la.start()
        for q in range(1, 4):
            xcopy(q, q % 2).wait()
            xq[pl.ds(q * Q, Q), :] = xtmp[q % 2].astype(jnp.float8_e4m3fn)
            if q + 2 < 4:
                xcopy(q + 2, q % 2).start()
            ra, la = hop0(q)
            ra.start()
            la.start()

        def wcopy(k, slot):
            return pltpu.make_async_copy(
                w_hbm.at[pl.ds(k * KC, KC), pl.ds(me * N_LOC, N_LOC)],
                wtmp.at[slot], wdma_sems.at[slot])

        wcopy(0, 0).start()
        wcopy(1, 1).start()
        for k in range(K // KC):
            wcopy(k, k % 2).wait()
            wq[pl.ds(k * KC, KC), :] = wtmp[k % 2].astype(jnp.float8_e5m2)
            if k + 2 < K // KC:
                wcopy(k + 2, k % 2).start()

        scale = sx_ref[0] * sw_ref[0]

        def mm(src, row0, n):
            acc = lax.dot_general(
                src[pl.ds(row0, n), :], wq[...],
                (((1,), (0,)), ((), ())),
                preferred_element_type=jnp.float32,
            )
            return acc * scale

        pending = [None, None]
        state = [0]

        def produce(origin, row0, n, src):
            slot = state[0] % 2
            state[0] += 1
            if pending[slot] is not None:
                pending[slot].wait()
            stage[slot, pl.ds(0, n), :] = mm(src, row0, n)
            cp = pltpu.make_async_copy(
                stage.at[slot, pl.ds(0, n), :],
                out_hbm.at[pl.ds(origin * M_LOC + row0, n), :],
                out_sems.at[slot])
            cp.start()
            pending[slot] = cp

        produce(me, 0, HALF, xq)
        produce(me, HALF, HALF, xq)

        recv_r = [rdma(rows(xq, q * Q, Q), rows(bufL, q * Q, Q), q, right)
                  for q in range(4)]
        recv_l = [rdma(rows(xq, q * Q, Q), rows(bufR, q * Q, Q), 4 + q, left)
                  for q in range(4)]

        recv_r[0].wait_recv()
        fwd_r[0].start()
        recv_r[1].wait_recv()
        fwd_r[1].start()
        fwd_r[2].start()
        recv_l[0].wait_recv()
        recv_l[1].wait_recv()
        produce(right, 0, HALF, bufR)
        recv_l[2].wait_recv()
        fwd_l[0].start()
        recv_l[3].wait_recv()
        fwd_l[1].start()
        fwd_l[2].start()
        produce(right, HALF, HALF, bufR)
        recv_r[2].wait_recv()
        recv_r[3].wait_recv()
        produce(left, 0, HALF, bufL)
        produce(left, HALF, HALF, bufL)

        fwd_r[0].wait_recv()
        produce(opp, 0, Q, bufO)
        fwd_l[0].wait_recv()
        produce(opp, HALF, Q, bufO)
        fwd_r[1].wait_recv()
        produce(opp, Q, E, bufO)
        fwd_l[1].wait_recv()
        produce(opp, HALF + Q, E, bufO)
        fwd_r[2].wait_recv()
        produce(opp, Q + E, E, bufO)
        fwd_l[2].wait_recv()
        produce(opp, HALF + Q + E, E, bufO)

        for q in range(4):
            recv_r[q].wait_send()
            recv_l[q].wait_send()
        for d in fwd_r:
            d.wait_send()
        for d in fwd_l:
            d.wait_send()
        pending[0].wait()
        pending[1].wait()

    fp8full = pltpu.VMEM((M_LOC, K), jnp.float8_e4m3fn)
    return pl.pallas_call(
        body,
        out_shape=jax.ShapeDtypeStruct((N_DEV * M_LOC, N_LOC), jnp.float32),
        in_specs=[
            pl.BlockSpec(memory_space=pl.ANY),
            pl.BlockSpec(memory_space=pl.ANY),
            pl.BlockSpec(memory_space=pltpu.SMEM),
            pl.BlockSpec(memory_space=pltpu.SMEM),
        ],
        out_specs=pl.BlockSpec(memory_space=pl.ANY),
        scratch_shapes=[
            fp8full,
            fp8full, fp8full, fp8full,
            pltpu.VMEM((K, N_LOC), jnp.float8_e5m2),
            pltpu.VMEM((2, KC, N_LOC), jnp.float32),
            pltpu.VMEM((2, Q, K), jnp.float32),
            pltpu.VMEM((2, HALF, N_LOC), jnp.float32),
            pltpu.SemaphoreType.DMA((14,)),
            pltpu.SemaphoreType.DMA((14,)),
            pltpu.SemaphoreType.DMA((2,)),
            pltpu.SemaphoreType.DMA((2,)),
            pltpu.SemaphoreType.DMA((2,)),
        ],
        compiler_params=pltpu.CompilerParams(
            collective_id=0,
            vmem_limit_bytes=100 * 1024 * 1024,
        ),
    )(x, w_mat, scale_x, scale_w)
